# baseline (speedup 1.0000x reference)
import jax
import jax.numpy as jnp
from jax import lax
from jax.experimental import pallas as pl
from jax.experimental.pallas import tpu as pltpu
from jax.experimental.pallas import tpu_sc as plsc

B = 16384
EMB = 32
K = 4
KD = EMB * K

NC = 2
NS = 16
NW = NC * NS
BPW = B // NW
CH = 128


def _sc_gather_body(uid_hbm, iid_hbm, uemb_hbm, iemb_hbm, ubias_hbm, ibias_hbm,
                    ue_out, ie_out, ub_out, ib_out,
                    ue_v, ie_v, ub_v, ib_v, uid_s, iid_s, sem):
    wid = lax.axis_index("s") * NC + lax.axis_index("c")
    base = wid * BPW
    pltpu.sync_copy(uid_hbm.at[pl.ds(base, BPW)], uid_s)
    pltpu.sync_copy(iid_hbm.at[pl.ds(base, BPW)], iid_s)

    def chunk(c, _):
        cbase = c * CH
        copies = []
        for g in range(CH // 16):
            uv = uid_s[pl.ds(cbase + g * 16, 16)]
            iv = iid_s[pl.ds(cbase + g * 16, 16)]
            for t in range(16):
                u = g * 16 + t
                ur = uv[t]
                ir = iv[t]
                copies.append(pltpu.make_async_copy(
                    uemb_hbm.at[pl.ds(ur, 1)], ue_v.at[pl.ds(u, 1)], sem))
                copies.append(pltpu.make_async_copy(
                    iemb_hbm.at[pl.ds(ir, 1)], ie_v.at[pl.ds(u, 1)], sem))
                copies.append(pltpu.make_async_copy(
                    ubias_hbm.at[pl.ds(ur, 1)], ub_v.at[pl.ds(u, 1)], sem))
                copies.append(pltpu.make_async_copy(
                    ibias_hbm.at[pl.ds(ir, 1)], ib_v.at[pl.ds(u, 1)], sem))
        for cp in copies:
            cp.start()
        for cp in copies:
            cp.wait()
        pltpu.sync_copy(ue_v, ue_out.at[pl.ds(base + cbase, CH)])
        pltpu.sync_copy(ie_v, ie_out.at[pl.ds(base + cbase, CH)])
        pltpu.sync_copy(ub_v, ub_out.at[pl.ds(base + cbase, CH)])
        pltpu.sync_copy(ib_v, ib_out.at[pl.ds(base + cbase, CH)])

    lax.fori_loop(0, BPW // CH, chunk, None)


def _sc_gather(uids, iids, uemb, iemb, ubias, ibias):
    mesh = plsc.VectorSubcoreMesh(core_axis_name="c", subcore_axis_name="s",
                                  num_cores=NC, num_subcores=NS)
    f = pl.kernel(
        _sc_gather_body,
        out_type=(
            jax.ShapeDtypeStruct((B, EMB), jnp.float32),
            jax.ShapeDtypeStruct((B, EMB), jnp.float32),
            jax.ShapeDtypeStruct((B, 1), jnp.float32),
            jax.ShapeDtypeStruct((B, 1), jnp.float32),
        ),
        mesh=mesh,
        compiler_params=pltpu.CompilerParams(use_tc_tiling_on_sc=True),
        scratch_types=[
            pltpu.VMEM((CH, EMB), jnp.float32),
            pltpu.VMEM((CH, EMB), jnp.float32),
            pltpu.VMEM((CH, 1), jnp.float32),
            pltpu.VMEM((CH, 1), jnp.float32),
            pltpu.VMEM((BPW,), jnp.int32),
            pltpu.VMEM((BPW,), jnp.int32),
            pltpu.SemaphoreType.DMA,
        ],
    )
    return f(uids, iids, uemb, iemb, ubias, ibias)


BLK = 2048


def _tc_mix_body(ue_ref, ie_ref, ub_ref, ib_ref, wt_ref, bt_ref, wa_ref, ba_ref,
                 out_ref):
    ue = ue_ref[...]
    ie = ie_ref[...]
    t = jnp.dot(ue, wt_ref[...], preferred_element_type=jnp.float32) + bt_ref[...]
    a = jnp.dot(ue, wa_ref[...], preferred_element_type=jnp.float32) + ba_ref[...]
    ier = jnp.concatenate([ie, ie, ie, ie], axis=1)
    q = t * ier
    l = a * ier
    ri = lax.broadcasted_iota(jnp.int32, (KD, KD), 0) // EMB
    ci = lax.broadcasted_iota(jnp.int32, (KD, KD), 1) // EMB
    s = (ri == ci).astype(jnp.float32)
    z = jnp.dot(l, s, preferred_element_type=jnp.float32)
    e = jnp.exp(z)
    denom = jnp.sum(e, axis=1, keepdims=True)
    num = jnp.sum(e * q, axis=1, keepdims=True)
    out_ref[...] = num * float(EMB) / denom + ub_ref[...] + ib_ref[...]


def _tc_mix(ue, ie, ub, ib, Wt, bt, Wa, ba):
    grid = (B // BLK,)
    emb = pl.BlockSpec((BLK, EMB), lambda i: (i, 0))
    col = pl.BlockSpec((BLK, 1), lambda i: (i, 0))
    w = pl.BlockSpec((EMB, KD), lambda i: (0, 0))
    bias = pl.BlockSpec((1, KD), lambda i: (0, 0))
    return pl.pallas_call(
        _tc_mix_body,
        grid=grid,
        in_specs=[emb, emb, col, col, w, bias, w, bias],
        out_specs=col,
        out_shape=jax.ShapeDtypeStruct((B, 1), jnp.float32),
    )(ue, ie, ub, ib, Wt, bt, Wa, ba)


@jax.jit
def kernel(user_ids, item_ids, user_emb, item_emb, user_bias, item_bias,
           Wt, bt, Wa, ba):
    uids = user_ids.astype(jnp.int32)
    iids = item_ids.astype(jnp.int32)
    ue, ie, ub, ib = _sc_gather(uids, iids, user_emb, item_emb,
                                user_bias, item_bias)
    out = _tc_mix(ue, ie, ub, ib, Wt, bt.reshape(1, KD), Wa, ba.reshape(1, KD))
    return out.reshape(-1)
